# Initial kernel scaffold; baseline (speedup 1.0000x reference)
import jax
import jax.numpy as jnp
from jax.experimental import pallas as pl


def kernel(points, normals, sa1_params, sa2_params, sa3_params, fc1_W, fc1_b):
    raise NotImplementedError("write your pallas kernel here")


# speedup vs baseline: 1.0995x; 1.0995x over previous
import functools

import jax
import jax.numpy as jnp
from jax.experimental import pallas as pl


def _fps_kernel(npoint, xyz_ref, out_ref):
    xt = xyz_ref[0]
    n = xt.shape[1]
    iota = jax.lax.broadcasted_iota(jnp.int32, (1, n), 1)
    iota_s = jax.lax.broadcasted_iota(jnp.int32, (1, npoint), 1)

    def body(i, state):
        dist, far, cent = state
        cent = jnp.where(iota_s == i, far, cent)
        mask = (iota == far).astype(jnp.float32)
        cx = jnp.sum(xt * mask, axis=1, keepdims=True)
        d = jnp.sum((xt - cx) ** 2, axis=0, keepdims=True)
        dist = jnp.minimum(dist, d)
        m = jnp.max(dist, axis=1, keepdims=True)
        far = jnp.min(jnp.where(dist == m, iota, n), axis=1)[0].astype(jnp.int32)
        return dist, far, cent

    dist0 = jnp.full((1, n), 1e10, dtype=jnp.float32)
    cent0 = jnp.zeros((1, npoint), dtype=jnp.int32)
    _, _, cent = jax.lax.fori_loop(0, npoint, body, (dist0, jnp.int32(0), cent0))
    out_ref[...] = jnp.broadcast_to(cent[None], out_ref.shape)


def _farthest_point_sample(xyz, npoint):
    B, N, _ = xyz.shape
    xt = jnp.transpose(xyz, (0, 2, 1))
    out = pl.pallas_call(
        functools.partial(_fps_kernel, npoint),
        grid=(B,),
        in_specs=[pl.BlockSpec((1, 3, N), lambda b: (b, 0, 0))],
        out_specs=pl.BlockSpec((1, 8, npoint), lambda b: (b, 0, 0)),
        out_shape=jax.ShapeDtypeStruct((B, 8, npoint), jnp.int32),
    )(xt)
    return out[:, 0, :]


def _mm_kernel(norm, x_ref, wt_ref, b_ref, mean_ref, inv_ref, gm_ref, bt_ref,
               y_ref, sum_ref, ssq_ref):
    x = x_ref[...]
    if norm:
        x = jnp.maximum((x - mean_ref[...]) * inv_ref[...] * gm_ref[...]
                        + bt_ref[...], 0.0)
    y = jnp.dot(x, wt_ref[...], preferred_element_type=jnp.float32)
    y = y + b_ref[...]
    y_ref[...] = y

    @pl.when(pl.program_id(0) == 0)
    def _():
        sum_ref[...] = jnp.zeros_like(sum_ref)
        ssq_ref[...] = jnp.zeros_like(ssq_ref)

    s = jnp.sum(y, axis=0, keepdims=True)
    q = jnp.sum(y * y, axis=0, keepdims=True)
    sum_ref[...] += jnp.broadcast_to(s, sum_ref.shape)
    ssq_ref[...] += jnp.broadcast_to(q, ssq_ref.shape)


def _mm_stats(x, wt, b, stats, norm):
    M, Cin = x.shape
    O = wt.shape[1]
    Mb = min(M, 1024)
    grid = M // Mb
    if stats is None:
        z = jnp.zeros((1, Cin), jnp.float32)
        mean_in, inv_in, gm_in, bt_in = z, z, z, z
    else:
        mean_in, inv_in, gm_in, bt_in = stats
    y, sums, ssqs = pl.pallas_call(
        functools.partial(_mm_kernel, norm),
        grid=(grid,),
        in_specs=[
            pl.BlockSpec((Mb, Cin), lambda i: (i, 0)),
            pl.BlockSpec((Cin, O), lambda i: (0, 0)),
            pl.BlockSpec((1, O), lambda i: (0, 0)),
            pl.BlockSpec((1, Cin), lambda i: (0, 0)),
            pl.BlockSpec((1, Cin), lambda i: (0, 0)),
            pl.BlockSpec((1, Cin), lambda i: (0, 0)),
            pl.BlockSpec((1, Cin), lambda i: (0, 0)),
        ],
        out_specs=(
            pl.BlockSpec((Mb, O), lambda i: (i, 0)),
            pl.BlockSpec((8, O), lambda i: (0, 0)),
            pl.BlockSpec((8, O), lambda i: (0, 0)),
        ),
        out_shape=(
            jax.ShapeDtypeStruct((M, O), jnp.float32),
            jax.ShapeDtypeStruct((8, O), jnp.float32),
            jax.ShapeDtypeStruct((8, O), jnp.float32),
        ),
    )(x, wt, b.reshape(1, O), mean_in, inv_in, gm_in, bt_in)
    mean = sums[0:1] / M
    var = ssqs[0:1] / M - mean * mean
    inv = jax.lax.rsqrt(var + 1e-5)
    return y, mean, inv


def _pool_kernel(x_ref, mean_ref, inv_ref, gm_ref, bt_ref, o_ref):
    x = x_ref[...]
    z = jnp.maximum((x - mean_ref[...][None]) * inv_ref[...][None]
                    * gm_ref[...][None] + bt_ref[...][None], 0.0)
    o_ref[...] = jnp.max(z, axis=1)


def _pool(y, mean, inv, gm, bt, BS, K):
    O = y.shape[-1]
    y3 = y.reshape(BS, K, O)
    R = max(1, min(BS, 4096 // K))
    while BS % R:
        R -= 1
    return pl.pallas_call(
        _pool_kernel,
        grid=(BS // R,),
        in_specs=[
            pl.BlockSpec((R, K, O), lambda i: (i, 0, 0)),
            pl.BlockSpec((1, O), lambda i: (0, 0)),
            pl.BlockSpec((1, O), lambda i: (0, 0)),
            pl.BlockSpec((1, O), lambda i: (0, 0)),
            pl.BlockSpec((1, O), lambda i: (0, 0)),
        ],
        out_specs=pl.BlockSpec((R, O), lambda i: (i, 0)),
        out_shape=jax.ShapeDtypeStruct((BS, O), jnp.float32),
    )(y3, mean, inv, gm.reshape(1, O), bt.reshape(1, O))


def _run_branch(g, convs):
    B, S, K, C = g.shape
    h = g.reshape(B * S * K, C)
    stats = None
    for li, (W, b, gm, bt) in enumerate(convs):
        h, mean, inv = _mm_stats(h, W.T, b, stats, norm=(li > 0))
        stats = (mean, inv, gm.reshape(1, -1), bt.reshape(1, -1))
    W, b, gm, bt = convs[-1]
    out = _pool(h, stats[0], stats[1], gm, bt, B * S, K)
    return out.reshape(B, S, -1)


def _index_points(points, idx):
    if idx.ndim == 2:
        return jnp.take_along_axis(points, idx[..., None], axis=1)
    B, S, K = idx.shape
    flat = idx.reshape(B, S * K)
    out = jnp.take_along_axis(points, flat[..., None], axis=1)
    return out.reshape(B, S, K, points.shape[-1])


def _query_ball_point(radius, nsample, xyz, new_xyz):
    B, N, _ = xyz.shape
    S = new_xyz.shape[1]
    sqrdists = jnp.sum(
        (new_xyz[:, :, None, :] - xyz[:, None, :, :]) ** 2, axis=-1)
    group_idx = jnp.broadcast_to(jnp.arange(N, dtype=jnp.int32), (B, S, N))
    group_idx = jnp.where(sqrdists > radius ** 2, N, group_idx)
    group_idx = jnp.sort(group_idx, axis=-1)[:, :, :nsample]
    group_first = jnp.broadcast_to(group_idx[:, :, :1], group_idx.shape)
    return jnp.where(group_idx == N, group_first, group_idx)


def _sa_msg(xyz, points, npoint, radius_list, nsample_list, branches):
    fps_idx = _farthest_point_sample(xyz, npoint)
    new_xyz = _index_points(xyz, fps_idx)
    outs = []
    for radius, K, convs in zip(radius_list, nsample_list, branches):
        gidx = _query_ball_point(radius, K, xyz, new_xyz)
        grouped_xyz = _index_points(xyz, gidx) - new_xyz[:, :, None, :]
        if points is not None:
            gp = _index_points(points, gidx)
            g = jnp.concatenate([gp, grouped_xyz], axis=-1)
        else:
            g = grouped_xyz
        outs.append(_run_branch(g, convs))
    return new_xyz, jnp.concatenate(outs, axis=-1)


def _fc_kernel(x_ref, wt_ref, b_ref, o_ref):
    y = jnp.dot(x_ref[...], wt_ref[...], preferred_element_type=jnp.float32)
    o_ref[...] = jnp.maximum(y + b_ref[...], 0.0)


def _fc_relu(x, W, b):
    M, Cin = x.shape
    O = W.shape[0]
    return pl.pallas_call(
        _fc_kernel,
        grid=(1,),
        in_specs=[
            pl.BlockSpec((M, Cin), lambda i: (0, 0)),
            pl.BlockSpec((Cin, O), lambda i: (0, 0)),
            pl.BlockSpec((1, O), lambda i: (0, 0)),
        ],
        out_specs=pl.BlockSpec((M, O), lambda i: (0, 0)),
        out_shape=jax.ShapeDtypeStruct((M, O), jnp.float32),
    )(x, W.T, b.reshape(1, O))


def kernel(points, normals, sa1_params, sa2_params, sa3_params, fc1_W, fc1_b):
    l1_xyz, l1_points = _sa_msg(points, normals, 512, [0.1, 0.2, 0.4],
                                [16, 32, 128], sa1_params)
    l2_xyz, l2_points = _sa_msg(l1_xyz, l1_points, 128, [0.2, 0.4, 0.8],
                                [32, 64, 128], sa2_params)
    B = l2_xyz.shape[0]
    g = jnp.concatenate([l2_xyz, l2_points], axis=-1)[:, None, :, :]
    x = _run_branch(g, sa3_params).reshape(B, -1)
    return _fc_relu(x, fc1_W, fc1_b)


# speedup vs baseline: 1.1496x; 1.0455x over previous
import functools

import jax
import jax.numpy as jnp
from jax.experimental import pallas as pl


def _fps_kernel(npoint, xyz_ref, out_ref):
    xt = xyz_ref[...]
    _, n, b = xt.shape
    iota = jax.lax.broadcasted_iota(jnp.int32, (n, b), 0)
    iota_col = jax.lax.broadcasted_iota(jnp.int32, (npoint, 1), 0)

    def body(i, state):
        dist, far = state
        out_ref[pl.ds(i, 1), :] = far
        mask = (iota == far).astype(jnp.float32)
        cx = jnp.sum(xt * mask[None], axis=1, keepdims=True)
        d = jnp.sum((xt - cx) ** 2, axis=0)
        dist = jnp.minimum(dist, d)
        m = jnp.max(dist, axis=0, keepdims=True)
        far = jnp.min(jnp.where(dist == m, iota, n), axis=0, keepdims=True)
        return dist, far

    dist0 = jnp.full((n, b), 1e10, dtype=jnp.float32)
    far0 = jnp.zeros((1, b), dtype=jnp.int32)
    jax.lax.fori_loop(0, npoint, body, (dist0, far0))


def _farthest_point_sample(xyz, npoint):
    B, N, _ = xyz.shape
    xt = jnp.transpose(xyz, (2, 1, 0))
    out = pl.pallas_call(
        functools.partial(_fps_kernel, npoint),
        grid=(1,),
        in_specs=[pl.BlockSpec((3, N, B), lambda i: (0, 0, 0))],
        out_specs=pl.BlockSpec((npoint, B), lambda i: (0, 0)),
        out_shape=jax.ShapeDtypeStruct((npoint, B), jnp.int32),
    )(xt)
    return out.T


def _mm_kernel(norm, x_ref, wt_ref, b_ref, mean_ref, inv_ref, gm_ref, bt_ref,
               y_ref, sum_ref, ssq_ref):
    x = x_ref[...]
    if norm:
        x = jnp.maximum((x - mean_ref[...]) * inv_ref[...] * gm_ref[...]
                        + bt_ref[...], 0.0)
    y = jnp.dot(x, wt_ref[...], preferred_element_type=jnp.float32)
    y = y + b_ref[...]
    y_ref[...] = y

    @pl.when(pl.program_id(0) == 0)
    def _():
        sum_ref[...] = jnp.zeros_like(sum_ref)
        ssq_ref[...] = jnp.zeros_like(ssq_ref)

    s = jnp.sum(y, axis=0, keepdims=True)
    q = jnp.sum(y * y, axis=0, keepdims=True)
    sum_ref[...] += jnp.broadcast_to(s, sum_ref.shape)
    ssq_ref[...] += jnp.broadcast_to(q, ssq_ref.shape)


def _mm_stats(x, wt, b, stats, norm):
    M, Cin = x.shape
    O = wt.shape[1]
    Mb = min(M, 1024)
    grid = M // Mb
    if stats is None:
        z = jnp.zeros((1, Cin), jnp.float32)
        mean_in, inv_in, gm_in, bt_in = z, z, z, z
    else:
        mean_in, inv_in, gm_in, bt_in = stats
    y, sums, ssqs = pl.pallas_call(
        functools.partial(_mm_kernel, norm),
        grid=(grid,),
        in_specs=[
            pl.BlockSpec((Mb, Cin), lambda i: (i, 0)),
            pl.BlockSpec((Cin, O), lambda i: (0, 0)),
            pl.BlockSpec((1, O), lambda i: (0, 0)),
            pl.BlockSpec((1, Cin), lambda i: (0, 0)),
            pl.BlockSpec((1, Cin), lambda i: (0, 0)),
            pl.BlockSpec((1, Cin), lambda i: (0, 0)),
            pl.BlockSpec((1, Cin), lambda i: (0, 0)),
        ],
        out_specs=(
            pl.BlockSpec((Mb, O), lambda i: (i, 0)),
            pl.BlockSpec((8, O), lambda i: (0, 0)),
            pl.BlockSpec((8, O), lambda i: (0, 0)),
        ),
        out_shape=(
            jax.ShapeDtypeStruct((M, O), jnp.float32),
            jax.ShapeDtypeStruct((8, O), jnp.float32),
            jax.ShapeDtypeStruct((8, O), jnp.float32),
        ),
    )(x, wt, b.reshape(1, O), mean_in, inv_in, gm_in, bt_in)
    mean = sums[0:1] / M
    var = ssqs[0:1] / M - mean * mean
    inv = jax.lax.rsqrt(var + 1e-5)
    return y, mean, inv


def _pool_kernel(x_ref, mean_ref, inv_ref, gm_ref, bt_ref, o_ref):
    x = x_ref[...]
    z = jnp.maximum((x - mean_ref[...][None]) * inv_ref[...][None]
                    * gm_ref[...][None] + bt_ref[...][None], 0.0)
    o_ref[...] = jnp.max(z, axis=1)


def _pool(y, mean, inv, gm, bt, BS, K):
    O = y.shape[-1]
    y3 = y.reshape(BS, K, O)
    R = max(1, min(BS, 4096 // K))
    while BS % R:
        R -= 1
    return pl.pallas_call(
        _pool_kernel,
        grid=(BS // R,),
        in_specs=[
            pl.BlockSpec((R, K, O), lambda i: (i, 0, 0)),
            pl.BlockSpec((1, O), lambda i: (0, 0)),
            pl.BlockSpec((1, O), lambda i: (0, 0)),
            pl.BlockSpec((1, O), lambda i: (0, 0)),
            pl.BlockSpec((1, O), lambda i: (0, 0)),
        ],
        out_specs=pl.BlockSpec((R, O), lambda i: (i, 0)),
        out_shape=jax.ShapeDtypeStruct((BS, O), jnp.float32),
    )(y3, mean, inv, gm.reshape(1, O), bt.reshape(1, O))


def _run_branch(g, convs):
    B, S, K, C = g.shape
    h = g.reshape(B * S * K, C)
    stats = None
    for li, (W, b, gm, bt) in enumerate(convs):
        h, mean, inv = _mm_stats(h, W.T, b, stats, norm=(li > 0))
        stats = (mean, inv, gm.reshape(1, -1), bt.reshape(1, -1))
    W, b, gm, bt = convs[-1]
    out = _pool(h, stats[0], stats[1], gm, bt, B * S, K)
    return out.reshape(B, S, -1)


def _index_points(points, idx):
    if idx.ndim == 2:
        return jnp.take_along_axis(points, idx[..., None], axis=1)
    B, S, K = idx.shape
    flat = idx.reshape(B, S * K)
    out = jnp.take_along_axis(points, flat[..., None], axis=1)
    return out.reshape(B, S, K, points.shape[-1])


def _query_ball_point(radius, nsample, xyz, new_xyz):
    B, N, _ = xyz.shape
    S = new_xyz.shape[1]
    sqrdists = jnp.sum(
        (new_xyz[:, :, None, :] - xyz[:, None, :, :]) ** 2, axis=-1)
    group_idx = jnp.broadcast_to(jnp.arange(N, dtype=jnp.int32), (B, S, N))
    group_idx = jnp.where(sqrdists > radius ** 2, N, group_idx)
    group_idx = jnp.sort(group_idx, axis=-1)[:, :, :nsample]
    group_first = jnp.broadcast_to(group_idx[:, :, :1], group_idx.shape)
    return jnp.where(group_idx == N, group_first, group_idx)


def _sa_msg(xyz, points, npoint, radius_list, nsample_list, branches):
    fps_idx = _farthest_point_sample(xyz, npoint)
    new_xyz = _index_points(xyz, fps_idx)
    outs = []
    for radius, K, convs in zip(radius_list, nsample_list, branches):
        gidx = _query_ball_point(radius, K, xyz, new_xyz)
        grouped_xyz = _index_points(xyz, gidx) - new_xyz[:, :, None, :]
        if points is not None:
            gp = _index_points(points, gidx)
            g = jnp.concatenate([gp, grouped_xyz], axis=-1)
        else:
            g = grouped_xyz
        outs.append(_run_branch(g, convs))
    return new_xyz, jnp.concatenate(outs, axis=-1)


def _fc_kernel(x_ref, wt_ref, b_ref, o_ref):
    y = jnp.dot(x_ref[...], wt_ref[...], preferred_element_type=jnp.float32)
    o_ref[...] = jnp.maximum(y + b_ref[...], 0.0)


def _fc_relu(x, W, b):
    M, Cin = x.shape
    O = W.shape[0]
    return pl.pallas_call(
        _fc_kernel,
        grid=(1,),
        in_specs=[
            pl.BlockSpec((M, Cin), lambda i: (0, 0)),
            pl.BlockSpec((Cin, O), lambda i: (0, 0)),
            pl.BlockSpec((1, O), lambda i: (0, 0)),
        ],
        out_specs=pl.BlockSpec((M, O), lambda i: (0, 0)),
        out_shape=jax.ShapeDtypeStruct((M, O), jnp.float32),
    )(x, W.T, b.reshape(1, O))


def kernel(points, normals, sa1_params, sa2_params, sa3_params, fc1_W, fc1_b):
    l1_xyz, l1_points = _sa_msg(points, normals, 512, [0.1, 0.2, 0.4],
                                [16, 32, 128], sa1_params)
    l2_xyz, l2_points = _sa_msg(l1_xyz, l1_points, 128, [0.2, 0.4, 0.8],
                                [32, 64, 128], sa2_params)
    B = l2_xyz.shape[0]
    g = jnp.concatenate([l2_xyz, l2_points], axis=-1)[:, None, :, :]
    x = _run_branch(g, sa3_params).reshape(B, -1)
    return _fc_relu(x, fc1_W, fc1_b)
